# baseline (speedup 1.0000x reference)
import jax
import jax.numpy as jnp
from jax import lax
from jax.experimental import pallas as pl
from jax.experimental.pallas import tpu as pltpu
from jax.experimental.pallas import tpu_sc as plsc

_B = 16384
_C = 1000
_NC = 2
_NS = 16
_NW = _NC * _NS
_RPW = _B // _NW
_CHUNK = 64
_NCHUNK = _RPW // _CHUNK
_L = 16


def _body(x_hbm, out_hbm, x_v, buf):
    wid = lax.axis_index("s") * _NC + lax.axis_index("c")
    base_row = wid * _RPW

    pltpu.sync_copy(x_hbm.at[pl.ds(base_row, _RPW)], x_v)

    zvec = jnp.zeros((_L,), jnp.float32)
    ones = jnp.ones((_L,), jnp.float32)
    iota = lax.iota(jnp.int32, _L)

    @pl.loop(0, _CHUNK)
    def _zero(r):
        for c in range(62):
            buf[r, pl.ds(c * _L, _L)] = zvec
        buf[r, pl.ds(_C - _L, _L)] = zvec

    def row_col(c, j):
        rows = c * _CHUNK + j * _L + iota
        cols = x_v[pl.ds(c * _CHUNK + j * _L, _L)]
        return rows % _CHUNK, cols

    for c in range(_NCHUNK):
        for j in range(_CHUNK // _L):
            rows = j * _L + iota
            cols = x_v[pl.ds(c * _CHUNK + j * _L, _L)]
            plsc.store_scatter(buf, [rows, cols], ones)
        pltpu.sync_copy(buf, out_hbm.at[pl.ds(base_row + c * _CHUNK, _CHUNK)])
        if c + 1 < _NCHUNK:
            for j in range(_CHUNK // _L):
                rows = j * _L + iota
                cols = x_v[pl.ds(c * _CHUNK + j * _L, _L)]
                plsc.store_scatter(buf, [rows, cols], zvec)


_onehot_sc = pl.kernel(
    _body,
    out_type=jax.ShapeDtypeStruct((_B, _C), jnp.float32),
    mesh=plsc.VectorSubcoreMesh(core_axis_name="c", subcore_axis_name="s"),
    scratch_types=[
        pltpu.VMEM((_RPW,), jnp.int32),
        pltpu.VMEM((_CHUNK, _C), jnp.float32),
    ],
    compiler_params=pltpu.CompilerParams(needs_layout_passes=False),
)


@jax.jit
def kernel(x):
    x = jnp.squeeze(x).astype(jnp.int32)
    return _onehot_sc(x)


# speedup vs baseline: 1.0004x; 1.0004x over previous
import jax
import jax.numpy as jnp
from jax import lax
from jax.experimental import pallas as pl
from jax.experimental.pallas import tpu as pltpu
from jax.experimental.pallas import tpu_sc as plsc

_B = 16384
_C = 1000
_NC = 2
_NS = 16
_NW = _NC * _NS
_RPW = _B // _NW
_CHUNK = 32
_NCHUNK = _RPW // _CHUNK
_L = 16


def _body(x_hbm, out_hbm, x_v, buf, s0, s1):
    wid = lax.axis_index("s") * _NC + lax.axis_index("c")
    base_row = wid * _RPW
    sems = (s0, s1)

    pltpu.sync_copy(x_hbm.at[pl.ds(base_row, _RPW)], x_v)

    zvec = jnp.zeros((_L,), jnp.float32)
    ones = jnp.ones((_L,), jnp.float32)
    iota = lax.iota(jnp.int32, _L)

    @pl.loop(0, 2 * _CHUNK)
    def _zero(r):
        for c in range(62):
            buf[r, pl.ds(c * _L, _L)] = zvec
        buf[r, pl.ds(_C - _L, _L)] = zvec

    def scatter(chunk, value):
        half = chunk % 2
        for j in range(_CHUNK // _L):
            rows = half * _CHUNK + j * _L + iota
            cols = x_v[pl.ds(chunk * _CHUNK + j * _L, _L)]
            plsc.store_scatter(buf, [rows, cols], value)

    copies = []
    for c in range(_NCHUNK):
        half = c % 2
        if c >= 2:
            copies[c - 2].wait()
            scatter(c - 2, zvec)
        scatter(c, ones)
        cp = pltpu.make_async_copy(
            buf.at[pl.ds(half * _CHUNK, _CHUNK)],
            out_hbm.at[pl.ds(base_row + c * _CHUNK, _CHUNK)],
            sems[half],
        )
        cp.start()
        copies.append(cp)
    copies[-2].wait()
    copies[-1].wait()


_onehot_sc = pl.kernel(
    _body,
    out_type=jax.ShapeDtypeStruct((_B, _C), jnp.float32),
    mesh=plsc.VectorSubcoreMesh(core_axis_name="c", subcore_axis_name="s"),
    scratch_types=[
        pltpu.VMEM((_RPW,), jnp.int32),
        pltpu.VMEM((2 * _CHUNK, _C), jnp.float32),
        pltpu.SemaphoreType.DMA,
        pltpu.SemaphoreType.DMA,
    ],
    compiler_params=pltpu.CompilerParams(needs_layout_passes=False),
)


@jax.jit
def kernel(x):
    x = jnp.squeeze(x).astype(jnp.int32)
    return _onehot_sc(x)


# speedup vs baseline: 1.0166x; 1.0162x over previous
import jax
import jax.numpy as jnp
from jax import lax
from jax.experimental import pallas as pl
from jax.experimental.pallas import tpu as pltpu
from jax.experimental.pallas import tpu_sc as plsc

_B = 16384
_C = 1000
_NC = 2
_NS = 16
_NW = _NC * _NS
_RPW = _B // _NW
_CHUNK = 32
_NCHUNK = _RPW // _CHUNK
_L = 16


def _body(x_hbm, out_hbm, x_v, buf, s0, s1, sx):
    wid = lax.axis_index("s") * _NC + lax.axis_index("c")
    base_row = wid * _RPW
    sems = (s0, s1)

    cp_x = pltpu.make_async_copy(x_hbm.at[pl.ds(base_row, _RPW)], x_v, sx)
    cp_x.start()

    zvec = jnp.zeros((_L,), jnp.float32)
    ones = jnp.ones((_L,), jnp.float32)
    iota = lax.iota(jnp.int32, _L)

    def memset_half(half):
        @pl.loop(half * _CHUNK, (half + 1) * _CHUNK)
        def _zero(r):
            for c in range(62):
                buf[r, pl.ds(c * _L, _L)] = zvec
            buf[r, pl.ds(_C - _L, _L)] = zvec

    def scatter(chunk, value):
        half = chunk % 2
        for j in range(_CHUNK // _L):
            rows = half * _CHUNK + j * _L + iota
            cols = x_v[pl.ds(chunk * _CHUNK + j * _L, _L)]
            plsc.store_scatter(buf, [rows, cols], value)

    def fire(chunk):
        half = chunk % 2
        cp = pltpu.make_async_copy(
            buf.at[pl.ds(half * _CHUNK, _CHUNK)],
            out_hbm.at[pl.ds(base_row + chunk * _CHUNK, _CHUNK)],
            sems[half],
        )
        cp.start()
        return cp

    memset_half(0)
    cp_x.wait()
    scatter(0, ones)
    copies = [fire(0)]
    memset_half(1)
    scatter(1, ones)
    copies.append(fire(1))
    for c in range(2, _NCHUNK):
        copies[c - 2].wait()
        scatter(c - 2, zvec)
        scatter(c, ones)
        copies.append(fire(c))
    copies[-2].wait()
    copies[-1].wait()


_onehot_sc = pl.kernel(
    _body,
    out_type=jax.ShapeDtypeStruct((_B, _C), jnp.float32),
    mesh=plsc.VectorSubcoreMesh(core_axis_name="c", subcore_axis_name="s"),
    scratch_types=[
        pltpu.VMEM((_RPW,), jnp.int32),
        pltpu.VMEM((2 * _CHUNK, _C), jnp.float32),
        pltpu.SemaphoreType.DMA,
        pltpu.SemaphoreType.DMA,
        pltpu.SemaphoreType.DMA,
    ],
    compiler_params=pltpu.CompilerParams(needs_layout_passes=False),
)


@jax.jit
def kernel(x):
    x = jnp.squeeze(x).astype(jnp.int32)
    return _onehot_sc(x)
